# baseline (speedup 1.0000x reference)
import functools

import jax
import jax.numpy as jnp
from jax import lax
from jax.experimental import pallas as pl
from jax.experimental.pallas import tpu as pltpu
from jax.experimental.pallas import tpu_sc as plsc

N = 10000
E = 320000
C = 128
T = 12

NSUB = 16
NCORES = 2
NW = NSUB * NCORES
KW = 80
ECHUNKS = (E // NSUB) // KW
ACHUNKS = (E // NW) // KW
NPS = N // NSUB
ROWS_BUF = 125


def _mm_body(x_ref, w_ref, o_ref):
    o_ref[...] = jnp.dot(x_ref[...], w_ref[...],
                         preferred_element_type=jnp.float32)


def _dis_body(degp_ref, dis_ref, selfn_ref):
    deg = degp_ref[0, :] + degp_ref[1, :] + 1.0
    dis_ref[...] = lax.rsqrt(deg)
    selfn_ref[...] = 1.0 / deg


def _deg_kernel(col_hbm, ew_hbm, degp_hbm, acc, zbuf, ew_v, col_v):
    c = lax.axis_index("c")
    s = lax.axis_index("s")
    wid = s * NCORES + c

    @pl.when(s == 0)
    def _():
        @pl.loop(0, 2000, step=16)
        def _(i):
            zbuf[pl.ds(i, 16)] = jnp.zeros((16,), jnp.float32)

        for kk in range(N // 2000):
            pltpu.sync_copy(zbuf, acc.at[pl.ds(kk * 2000, 2000)])

    plsc.subcore_barrier()

    pltpu.sync_copy(ew_hbm.at[pl.ds(pl.multiple_of(wid * (E // NW), 8), E // NW)],
                    ew_v)
    pltpu.sync_copy(col_hbm.at[wid], col_v)

    @pl.loop(0, ACHUNKS)
    def _(cc):
        pltpu.sync_copy(ew_v.at[pl.ds(pl.multiple_of(cc * KW, 8), KW)],
                        acc.at[col_v.at[cc]], add=True)

    plsc.subcore_barrier()

    @pl.when(s == 0)
    def _():
        pltpu.sync_copy(acc, degp_hbm.at[c])


def _norm_kernel(recA_hbm, dis_hbm, nrm_hbm, dis_v, rec_v, nrm_v, rsem, wsem):
    c = lax.axis_index("c")
    s = lax.axis_index("s")
    wid = s * NCORES + c

    pltpu.sync_copy(dis_hbm, dis_v)

    def rstart(i, b):
        pltpu.async_copy(recA_hbm.at[wid].at[i], rec_v.at[b], rsem.at[b])

    def rwait(i, b):
        pltpu.make_async_copy(recA_hbm.at[wid].at[i], rec_v.at[b],
                              rsem.at[b]).wait()

    def wstart(i, b):
        pltpu.async_copy(nrm_v.at[b], nrm_hbm.at[wid].at[i], wsem.at[b])

    def wwait(i, b):
        pltpu.make_async_copy(nrm_v.at[b], nrm_hbm.at[wid].at[i],
                              wsem.at[b]).wait()

    def compute(b):
        for k in range(KW // 16):
            r16 = rec_v[b, 0, pl.ds(k * 16, 16)]
            c16 = rec_v[b, 1, pl.ds(k * 16, 16)]
            ew16 = plsc.bitcast(rec_v[b, 2, pl.ds(k * 16, 16)], jnp.float32)
            nr = plsc.load_gather(dis_v, [r16])
            nc = plsc.load_gather(dis_v, [c16])
            nrm_v[b, pl.ds(k * 16, 16)] = nr * ew16 * nc

    rstart(0, 0)

    @pl.loop(0, ACHUNKS - 1, step=2)
    def _(i0):
        for b in (0, 1):
            i = i0 + b
            o = 1 - b
            rstart(i + 1, o)
            rwait(i, b)

            @pl.when(i > 1)
            def _():
                wwait(i - 2, b)

            compute(b)
            wstart(i, b)

    last = ACHUNKS - 1
    rwait(last, 0)
    wwait(last - 2, 0)
    compute(0)
    wstart(last, 0)
    wwait(last - 1, 1)
    wwait(last, 0)


def _msg_kernel(h_hbm, idxt_hbm, rec_hbm, selfn_hbm, b_hbm, out_hbm,
                acc, selfn_v, b_v, buf0, msg_v, rec_v, idx_v,
                gsem, ssem, rsem, isem):
    c = lax.axis_index("c")
    s = lax.axis_index("s")

    pltpu.sync_copy(selfn_hbm.at[s], selfn_v)
    pltpu.sync_copy(b_hbm, b_v)

    r0 = s * NPS
    t0 = c * (T // NCORES)

    def rec_start(i, b):
        pltpu.async_copy(rec_hbm.at[s].at[i], rec_v.at[b], rsem.at[b])

    def rec_wait(i, b):
        pltpu.make_async_copy(rec_hbm.at[s].at[i], rec_v.at[b],
                              rsem.at[b]).wait()

    def idx_start(t, i, b):
        pltpu.async_copy(idxt_hbm.at[t].at[s].at[i], idx_v.at[b], isem.at[b])

    def idx_wait(t, i, b):
        pltpu.make_async_copy(idxt_hbm.at[t].at[s].at[i], idx_v.at[b],
                              isem.at[b]).wait()

    def gather_start(b):
        pltpu.async_copy(h_hbm.at[idx_v.at[b]], msg_v.at[b], gsem.at[b])

    def gather_wait(b):
        pltpu.make_async_copy(h_hbm.at[idx_v.at[b]], msg_v.at[b],
                              gsem.at[b]).wait()

    def scale(b):
        @plsc.parallel_loop(0, KW, step=1, unroll=8)
        def _(e):
            sp = plsc.bitcast(
                plsc.load_gather(rec_v.at[b], [
                    jnp.zeros((16,), jnp.int32) + 1,
                    jnp.zeros((16,), jnp.int32) + e]), jnp.float32)
            for k in range(C // 16):
                msg_v[b, e, pl.ds(k * 16, 16)] = (
                    msg_v[b, e, pl.ds(k * 16, 16)] * sp)

    def scat_start(b):
        pltpu.async_copy(msg_v.at[b], acc.at[rec_v.at[b].at[0]], ssem.at[b],
                         add=True)

    def scat_wait(b):
        pltpu.make_async_copy(msg_v.at[b], acc.at[rec_v.at[b].at[0]],
                              ssem.at[b]).wait()

    @pl.loop(0, T // NCORES)
    def _(ti):
        t = t0 + ti
        base = pl.multiple_of(t * N, 8)

        idx_start(t, 0, 0)
        idx_start(t, 1, 1)
        rec_start(0, 0)

        @pl.loop(0, NPS // ROWS_BUF)
        def _(cb):
            off = r0 + cb * ROWS_BUF
            pltpu.sync_copy(h_hbm.at[pl.ds(base + off, ROWS_BUF)], buf0)

            @plsc.parallel_loop(0, ROWS_BUF, step=1, unroll=4)
            def _(j):
                sp = plsc.load_gather(
                    selfn_v, [jnp.zeros((16,), jnp.int32) + (cb * ROWS_BUF + j)])
                for k in range(C // 16):
                    buf0[j, pl.ds(k * 16, 16)] = buf0[j, pl.ds(k * 16, 16)] * sp

            pltpu.sync_copy(buf0, acc.at[pl.ds(off, ROWS_BUF)])

        idx_wait(t, 0, 0)
        gather_start(0)
        plsc.subcore_barrier()

        @pl.loop(0, ECHUNKS, step=2)
        def _(i0):
            for b in (0, 1):
                i = i0 + b
                o = 1 - b

                @pl.when(i > 0)
                def _():
                    scat_wait(o)

                @pl.when(i + 1 < ECHUNKS)
                def _():
                    rec_start(i + 1, o)

                gather_wait(b)

                @pl.when(i + 2 < ECHUNKS)
                def _():
                    idx_start(t, i + 2, b)

                @pl.when(i + 1 < ECHUNKS)
                def _():
                    idx_wait(t, i + 1, o)
                    gather_start(o)

                rec_wait(i, b)
                scale(b)
                scat_start(b)

        scat_wait((ECHUNKS - 1) % 2)
        plsc.subcore_barrier()

        @pl.loop(0, NPS // ROWS_BUF)
        def _(cb):
            off = r0 + cb * ROWS_BUF
            pltpu.sync_copy(acc.at[pl.ds(off, ROWS_BUF)], buf0)

            @plsc.parallel_loop(0, ROWS_BUF, step=1, unroll=4)
            def _(j):
                for k in range(C // 16):
                    v = buf0[j, pl.ds(k * 16, 16)] + b_v[pl.ds(k * 16, 16)]
                    buf0[j, pl.ds(k * 16, 16)] = jnp.maximum(v, 0.0)

            pltpu.sync_copy(buf0, out_hbm.at[pl.ds(base + off, ROWS_BUF)])

        plsc.subcore_barrier()


@jax.jit
def kernel(x, edge_index, edge_attr, W, b):
    col = edge_index[1]

    x2 = jnp.transpose(x, (2, 0, 1)).reshape(T * N, C)

    h = pl.pallas_call(
        _mm_body,
        grid=(T * N // 2000,),
        in_specs=[pl.BlockSpec((2000, C), lambda i: (i, 0)),
                  pl.BlockSpec((C, C), lambda i: (0, 0))],
        out_specs=pl.BlockSpec((2000, C), lambda i: (i, 0)),
        out_shape=jax.ShapeDtypeStruct((T * N, C), jnp.float32),
    )(x2, W)

    mesh = plsc.VectorSubcoreMesh(core_axis_name="c", subcore_axis_name="s")
    sc_params = pltpu.CompilerParams(use_tc_tiling_on_sc=False,
                                     needs_layout_passes=False)

    deg_call = functools.partial(
        pl.kernel,
        out_type=jax.ShapeDtypeStruct((NCORES, N), jnp.float32),
        mesh=mesh,
        compiler_params=sc_params,
        scratch_types=[
            pltpu.VMEM_SHARED((N,), jnp.float32),
            pltpu.VMEM((2000,), jnp.float32),
            pltpu.VMEM((E // NW,), jnp.float32),
            pltpu.VMEM((ACHUNKS, KW), jnp.int32),
        ],
    )
    degp = deg_call(_deg_kernel)(col.reshape(NW, ACHUNKS, KW), edge_attr)

    dis, selfn = pl.pallas_call(
        _dis_body,
        out_shape=(jax.ShapeDtypeStruct((N,), jnp.float32),
                   jax.ShapeDtypeStruct((N,), jnp.float32)),
    )(degp)

    ew_bits = lax.bitcast_convert_type(edge_attr, jnp.int32)
    rec_a = jnp.concatenate([edge_index, ew_bits[None, :]], axis=0)
    rec_a = rec_a.reshape(3, NW, ACHUNKS, KW).transpose(1, 2, 0, 3)
    norm_call = functools.partial(
        pl.kernel,
        out_type=jax.ShapeDtypeStruct((NW, ACHUNKS, KW), jnp.float32),
        mesh=mesh,
        compiler_params=sc_params,
        scratch_types=[
            pltpu.VMEM((N,), jnp.float32),
            pltpu.VMEM((2, 3, KW), jnp.int32),
            pltpu.VMEM((2, KW), jnp.float32),
            pltpu.SemaphoreType.DMA((2,)),
            pltpu.SemaphoreType.DMA((2,)),
        ],
    )
    nrm = norm_call(_norm_kernel)(rec_a, dis)

    nrm_bits = lax.bitcast_convert_type(nrm.reshape(E), jnp.int32)
    rec = jnp.stack([edge_index[1], nrm_bits])
    rec_c = rec.reshape(2, NSUB, ECHUNKS, KW).transpose(1, 2, 0, 3)
    idxt = (edge_index[0][None, :]
            + (jnp.arange(T, dtype=jnp.int32) * N)[:, None])
    idxt = idxt.reshape(T, NSUB, ECHUNKS, KW)

    msg_call = functools.partial(
        pl.kernel,
        out_type=jax.ShapeDtypeStruct((T * N, C), jnp.float32),
        mesh=mesh,
        compiler_params=sc_params,
        scratch_types=[
            pltpu.VMEM_SHARED((N, C), jnp.float32),
            pltpu.VMEM((NPS,), jnp.float32),
            pltpu.VMEM((C,), jnp.float32),
            pltpu.VMEM((ROWS_BUF, C), jnp.float32),
            pltpu.VMEM((2, KW, C), jnp.float32),
            pltpu.VMEM((2, 2, KW), jnp.int32),
            pltpu.VMEM((2, KW), jnp.int32),
            pltpu.SemaphoreType.DMA((2,)),
            pltpu.SemaphoreType.DMA((2,)),
            pltpu.SemaphoreType.DMA((2,)),
            pltpu.SemaphoreType.DMA((2,)),
        ],
    )
    outf = msg_call(_msg_kernel)(
        h, idxt, rec_c, selfn.reshape(NSUB, NPS), b)

    return outf.reshape(T, N, C).transpose(1, 2, 0)


# speedup vs baseline: 1.2074x; 1.2074x over previous
import functools

import jax
import jax.numpy as jnp
import numpy as np
from jax import lax
from jax.experimental import pallas as pl
from jax.experimental.pallas import tpu as pltpu
from jax.experimental.pallas import tpu_sc as plsc

N = 10000
E = 320000
C = 128
T = 12

NSUB = 16
NCORES = 2
NW = NSUB * NCORES
KW = 80
ACHUNKS = (E // NW) // KW
NPS = N // NSUB
ROWS_BUF = 125
EPS = E // NSUB
EPSX = EPS + NPS + 15
ECX = EPSX // KW

_LO_COLS = np.concatenate(
    [np.arange(g * 32, g * 32 + 16) for g in range(C // 32)]).astype(np.int32)
_HI_COLS = _LO_COLS + 16


def _rne_bf16_bits(y):
    v = lax.bitcast_convert_type(y, jnp.int32)
    return v + (jnp.int32(0x7FFF) + ((v >> 16) & 1))


def _mm_body(x_ref, wlo_ref, whi_ref, o_ref):
    x = x_ref[...]
    ylo = jnp.dot(x, wlo_ref[...], preferred_element_type=jnp.float32)
    yhi = jnp.dot(x, whi_ref[...], preferred_element_type=jnp.float32)
    lo = (_rne_bf16_bits(ylo) >> 16) & jnp.int32(0xFFFF)
    hi = _rne_bf16_bits(yhi) & jnp.int32(-65536)
    o_ref[...] = lo | hi


def _dis_body(degp_ref, dis_ref, selfn_ref):
    deg = degp_ref[0, :] + degp_ref[1, :] + 1.0
    dis_ref[...] = lax.rsqrt(deg)
    selfn_ref[...] = 1.0 / deg


def _deg_kernel(col_hbm, ew_hbm, degp_hbm, acc, zbuf, ew_v, col_v):
    c = lax.axis_index("c")
    s = lax.axis_index("s")
    wid = s * NCORES + c

    @pl.when(s == 0)
    def _():
        @pl.loop(0, 2000, step=16)
        def _(i):
            zbuf[pl.ds(i, 16)] = jnp.zeros((16,), jnp.float32)

        for kk in range(N // 2000):
            pltpu.sync_copy(zbuf, acc.at[pl.ds(kk * 2000, 2000)])

    plsc.subcore_barrier()

    pltpu.sync_copy(ew_hbm.at[pl.ds(pl.multiple_of(wid * (E // NW), 8), E // NW)],
                    ew_v)
    pltpu.sync_copy(col_hbm.at[wid], col_v)

    @pl.loop(0, ACHUNKS)
    def _(cc):
        pltpu.sync_copy(ew_v.at[pl.ds(pl.multiple_of(cc * KW, 8), KW)],
                        acc.at[col_v.at[cc]], add=True)

    plsc.subcore_barrier()

    @pl.when(s == 0)
    def _():
        pltpu.sync_copy(acc, degp_hbm.at[c])


def _norm_kernel(recA_hbm, dis_hbm, nrm_hbm, dis_v, rec_v, nrm_v, rsem, wsem):
    c = lax.axis_index("c")
    s = lax.axis_index("s")
    wid = s * NCORES + c

    pltpu.sync_copy(dis_hbm, dis_v)

    def rstart(i, b):
        pltpu.async_copy(recA_hbm.at[wid].at[i], rec_v.at[b], rsem.at[b])

    def rwait(i, b):
        pltpu.make_async_copy(recA_hbm.at[wid].at[i], rec_v.at[b],
                              rsem.at[b]).wait()

    def wstart(i, b):
        pltpu.async_copy(nrm_v.at[b], nrm_hbm.at[wid].at[i], wsem.at[b])

    def wwait(i, b):
        pltpu.make_async_copy(nrm_v.at[b], nrm_hbm.at[wid].at[i],
                              wsem.at[b]).wait()

    def compute(b):
        for k in range(KW // 16):
            r16 = rec_v[b, 0, pl.ds(k * 16, 16)]
            c16 = rec_v[b, 1, pl.ds(k * 16, 16)]
            ew16 = plsc.bitcast(rec_v[b, 2, pl.ds(k * 16, 16)], jnp.float32)
            nr = plsc.load_gather(dis_v, [r16])
            nc = plsc.load_gather(dis_v, [c16])
            nrm_v[b, pl.ds(k * 16, 16)] = nr * ew16 * nc

    rstart(0, 0)

    @pl.loop(0, ACHUNKS - 1, step=2)
    def _(i0):
        for b in (0, 1):
            i = i0 + b
            o = 1 - b
            rstart(i + 1, o)
            rwait(i, b)

            @pl.when(i > 1)
            def _():
                wwait(i - 2, b)

            compute(b)
            wstart(i, b)

    last = ACHUNKS - 1
    rwait(last, 0)
    wwait(last - 2, 0)
    compute(0)
    wstart(last, 0)
    wwait(last - 1, 1)
    wwait(last, 0)


def _msg_kernel(h_hbm, idxt_hbm, rec_hbm, b_hbm, out_hbm,
                acc, b_v, buf0, msg_v, fbuf, rec_v, idx_v,
                gsem, ssem, rsem, isem):
    c = lax.axis_index("c")
    s = lax.axis_index("s")

    pltpu.sync_copy(b_hbm, b_v)

    r0 = s * NPS
    t0 = c * (T // NCORES)

    def rec_start(i, b):
        pltpu.async_copy(rec_hbm.at[s].at[i], rec_v.at[b], rsem.at[b])

    def rec_wait(i, b):
        pltpu.make_async_copy(rec_hbm.at[s].at[i], rec_v.at[b],
                              rsem.at[b]).wait()

    def idx_start(t, i, b):
        pltpu.async_copy(idxt_hbm.at[t].at[s].at[i], idx_v.at[b], isem.at[b])

    def idx_wait(t, i, b):
        pltpu.make_async_copy(idxt_hbm.at[t].at[s].at[i], idx_v.at[b],
                              isem.at[b]).wait()

    def gather_start(b):
        pltpu.async_copy(h_hbm.at[idx_v.at[b]], msg_v.at[b], gsem.at[b])

    def gather_wait(b):
        pltpu.make_async_copy(h_hbm.at[idx_v.at[b]], msg_v.at[b],
                              gsem.at[b]).wait()

    def scale(b):
        @plsc.parallel_loop(0, KW, step=1, unroll=8)
        def _(e):
            sp = plsc.bitcast(
                plsc.load_gather(rec_v.at[b], [
                    jnp.zeros((16,), jnp.int32) + 1,
                    jnp.zeros((16,), jnp.int32) + e]), jnp.float32)
            for k in range(C // 32):
                xi = msg_v[b, e, pl.ds(k * 16, 16)]
                fe = plsc.bitcast(xi << 16, jnp.float32) * sp
                fo = plsc.bitcast(xi & jnp.int32(-65536), jnp.float32) * sp
                fbuf[b, e, pl.ds(k * 32, 16)] = fe
                fbuf[b, e, pl.ds(k * 32 + 16, 16)] = fo

    def scat_start(b):
        pltpu.async_copy(fbuf.at[b], acc.at[rec_v.at[b].at[0]], ssem.at[b],
                         add=True)

    def scat_wait(b):
        pltpu.make_async_copy(fbuf.at[b], acc.at[rec_v.at[b].at[0]],
                              ssem.at[b]).wait()

    @pl.loop(0, T // NCORES)
    def _(ti):
        t = t0 + ti
        base = pl.multiple_of(t * N, 8)

        idx_start(t, 0, 0)
        idx_start(t, 1, 1)
        rec_start(0, 0)

        @plsc.parallel_loop(0, ROWS_BUF, step=1, unroll=4)
        def _(j):
            for k in range(C // 16):
                buf0[j, pl.ds(k * 16, 16)] = jnp.zeros((16,), jnp.float32)

        @pl.loop(0, NPS // ROWS_BUF)
        def _(cb):
            pltpu.sync_copy(buf0, acc.at[pl.ds(r0 + cb * ROWS_BUF, ROWS_BUF)])

        idx_wait(t, 0, 0)
        gather_start(0)
        plsc.subcore_barrier()

        @pl.loop(0, ECX, step=2)
        def _(i0):
            for b in (0, 1):
                i = i0 + b
                o = 1 - b

                @pl.when(i > 0)
                def _():
                    scat_wait(o)

                @pl.when(i + 1 < ECX)
                def _():
                    rec_start(i + 1, o)

                gather_wait(b)

                @pl.when(i + 2 < ECX)
                def _():
                    idx_start(t, i + 2, b)

                @pl.when(i + 1 < ECX)
                def _():
                    idx_wait(t, i + 1, o)
                    gather_start(o)

                rec_wait(i, b)
                scale(b)
                scat_start(b)

        scat_wait((ECX - 1) % 2)
        plsc.subcore_barrier()

        @pl.loop(0, NPS // ROWS_BUF)
        def _(cb):
            off = r0 + cb * ROWS_BUF
            pltpu.sync_copy(acc.at[pl.ds(off, ROWS_BUF)], buf0)

            @plsc.parallel_loop(0, ROWS_BUF, step=1, unroll=4)
            def _(j):
                for k in range(C // 16):
                    v = buf0[j, pl.ds(k * 16, 16)] + b_v[pl.ds(k * 16, 16)]
                    buf0[j, pl.ds(k * 16, 16)] = jnp.maximum(v, 0.0)

            pltpu.sync_copy(buf0, out_hbm.at[pl.ds(base + off, ROWS_BUF)])

        plsc.subcore_barrier()


@jax.jit
def kernel(x, edge_index, edge_attr, W, b):
    col = edge_index[1]

    x2 = jnp.transpose(x, (2, 0, 1)).reshape(T * N, C)

    h = pl.pallas_call(
        _mm_body,
        grid=(T * N // 2000,),
        in_specs=[pl.BlockSpec((2000, C), lambda i: (i, 0)),
                  pl.BlockSpec((C, C // 2), lambda i: (0, 0)),
                  pl.BlockSpec((C, C // 2), lambda i: (0, 0))],
        out_specs=pl.BlockSpec((2000, C // 2), lambda i: (i, 0)),
        out_shape=jax.ShapeDtypeStruct((T * N, C // 2), jnp.int32),
    )(x2, W[:, _LO_COLS], W[:, _HI_COLS])

    mesh = plsc.VectorSubcoreMesh(core_axis_name="c", subcore_axis_name="s")
    sc_params = pltpu.CompilerParams(use_tc_tiling_on_sc=False,
                                     needs_layout_passes=False)

    deg_call = functools.partial(
        pl.kernel,
        out_type=jax.ShapeDtypeStruct((NCORES, N), jnp.float32),
        mesh=mesh,
        compiler_params=sc_params,
        scratch_types=[
            pltpu.VMEM_SHARED((N,), jnp.float32),
            pltpu.VMEM((2000,), jnp.float32),
            pltpu.VMEM((E // NW,), jnp.float32),
            pltpu.VMEM((ACHUNKS, KW), jnp.int32),
        ],
    )
    degp = deg_call(_deg_kernel)(col.reshape(NW, ACHUNKS, KW), edge_attr)

    dis, selfn = pl.pallas_call(
        _dis_body,
        out_shape=(jax.ShapeDtypeStruct((N,), jnp.float32),
                   jax.ShapeDtypeStruct((N,), jnp.float32)),
    )(degp)

    ew_bits = lax.bitcast_convert_type(edge_attr, jnp.int32)
    rec_a = jnp.concatenate([edge_index, ew_bits[None, :]], axis=0)
    rec_a = rec_a.reshape(3, NW, ACHUNKS, KW).transpose(1, 2, 0, 3)
    norm_call = functools.partial(
        pl.kernel,
        out_type=jax.ShapeDtypeStruct((NW, ACHUNKS, KW), jnp.float32),
        mesh=mesh,
        compiler_params=sc_params,
        scratch_types=[
            pltpu.VMEM((N,), jnp.float32),
            pltpu.VMEM((2, 3, KW), jnp.int32),
            pltpu.VMEM((2, KW), jnp.float32),
            pltpu.SemaphoreType.DMA((2,)),
            pltpu.SemaphoreType.DMA((2,)),
        ],
    )
    nrm = norm_call(_norm_kernel)(rec_a, dis)

    row = edge_index[0]
    selfsrc = jnp.arange(N, dtype=jnp.int32).reshape(NSUB, NPS)
    zi = jnp.zeros((NSUB, EPSX - EPS - NPS), jnp.int32)
    zf = jnp.zeros((NSUB, EPSX - EPS - NPS), jnp.float32)
    row_x = jnp.concatenate([row.reshape(NSUB, EPS), selfsrc, zi], axis=1)
    col_x = jnp.concatenate([col.reshape(NSUB, EPS), selfsrc, zi], axis=1)
    nrm_x = jnp.concatenate(
        [nrm.reshape(NSUB, EPS), selfn.reshape(NSUB, NPS), zf], axis=1)
    rec = jnp.stack([col_x, lax.bitcast_convert_type(nrm_x, jnp.int32)])
    rec_c = rec.reshape(2, NSUB, ECX, KW).transpose(1, 2, 0, 3)
    idxt = (row_x[None]
            + (jnp.arange(T, dtype=jnp.int32) * N)[:, None, None])
    idxt = idxt.reshape(T, NSUB, ECX, KW)

    msg_call = functools.partial(
        pl.kernel,
        out_type=jax.ShapeDtypeStruct((T * N, C), jnp.float32),
        mesh=mesh,
        compiler_params=sc_params,
        scratch_types=[
            pltpu.VMEM_SHARED((N, C), jnp.float32),
            pltpu.VMEM((C,), jnp.float32),
            pltpu.VMEM((ROWS_BUF, C), jnp.float32),
            pltpu.VMEM((2, KW, C // 2), jnp.int32),
            pltpu.VMEM((2, KW, C), jnp.float32),
            pltpu.VMEM((2, 2, KW), jnp.int32),
            pltpu.VMEM((2, KW), jnp.int32),
            pltpu.SemaphoreType.DMA((2,)),
            pltpu.SemaphoreType.DMA((2,)),
            pltpu.SemaphoreType.DMA((2,)),
            pltpu.SemaphoreType.DMA((2,)),
        ],
    )
    outf = msg_call(_msg_kernel)(h, idxt, rec_c, b)

    return outf.reshape(T, N, C).transpose(1, 2, 0)
